# Initial kernel scaffold; baseline (speedup 1.0000x reference)
import jax
import jax.numpy as jnp
from jax.experimental import pallas as pl


def kernel(inputs, word_table, pos_table):
    raise NotImplementedError("write your pallas kernel here")


# speedup vs baseline: 1.4155x; 1.4155x over previous
import functools

import jax
import jax.numpy as jnp
from jax import lax
from jax.experimental import pallas as pl
from jax.experimental.pallas import tpu as pltpu
from jax.experimental.pallas import tpu_sc as plsc

SEQ = 200
DIM = 32
BATCH = 4096

_NC = 2
_NS = 16
_NW = _NC * _NS

ROWS_PER_W = (BATCH * SEQ) // _NW
CHUNK_BATCH = 4
CHUNK = CHUNK_BATCH * SEQ
N_CHUNKS = ROWS_PER_W // CHUNK


def _emb_body(idx_hbm, table_hbm, pos_hbm, out_hbm, idx_v, rows_v, pos_v, sem):
    wid = lax.axis_index("s") * _NC + lax.axis_index("c")
    base = wid * ROWS_PER_W

    pltpu.sync_copy(pos_hbm, pos_v)
    pltpu.sync_copy(idx_hbm.at[pl.ds(base, ROWS_PER_W)], idx_v)

    def chunk_body(g, carry):
        off = g * CHUNK
        pltpu.async_copy(
            table_hbm.at[idx_v.at[pl.ds(off, CHUNK)]], rows_v, sem
        ).wait()

        def s_body(s, c):
            p0 = pos_v[s, pl.ds(0, 16)]
            p1 = pos_v[s, pl.ds(16, 16)]
            for r in range(CHUNK_BATCH):
                q = r * SEQ + s
                rows_v[q, pl.ds(0, 16)] = rows_v[q, pl.ds(0, 16)] + p0
                rows_v[q, pl.ds(16, 16)] = rows_v[q, pl.ds(16, 16)] + p1
            return c

        lax.fori_loop(0, SEQ, s_body, 0)
        pltpu.sync_copy(rows_v, out_hbm.at[pl.ds(base + off, CHUNK)])
        return carry

    lax.fori_loop(0, N_CHUNKS, chunk_body, 0)


_emb = functools.partial(
    pl.kernel,
    mesh=plsc.VectorSubcoreMesh(core_axis_name="c", subcore_axis_name="s"),
    out_type=jax.ShapeDtypeStruct((BATCH * SEQ, DIM), jnp.float32),
    scratch_types=[
        pltpu.VMEM((ROWS_PER_W,), jnp.int32),
        pltpu.VMEM((CHUNK, DIM), jnp.float32),
        pltpu.VMEM((SEQ, DIM), jnp.float32),
        pltpu.SemaphoreType.DMA,
    ],
    compiler_params=pltpu.CompilerParams(use_tc_tiling_on_sc=False),
)(_emb_body)


def kernel(inputs, word_table, pos_table):
    flat_idx = inputs.reshape(-1).astype(jnp.int32)
    out = _emb(flat_idx, word_table, pos_table)
    return out.reshape(inputs.shape[0], inputs.shape[1], DIM)


# speedup vs baseline: 1.9260x; 1.3607x over previous
import functools

import jax
import jax.numpy as jnp
from jax import lax
from jax.experimental import pallas as pl
from jax.experimental.pallas import tpu as pltpu
from jax.experimental.pallas import tpu_sc as plsc

SEQ = 200
DIM = 32
BATCH = 4096
VOCAB = 1000000

_NC = 2
_NS = 16
_NW = _NC * _NS

ROWS_PER_W = (BATCH * SEQ) // _NW
CHUNK_BATCH = 4
CHUNK = CHUNK_BATCH * SEQ
N_CHUNKS = ROWS_PER_W // CHUNK


def _emb_body(idx_hbm, table_hbm, pos_hbm, out_hbm, idx_v, rows_v, pos_v, sem):
    wid = lax.axis_index("s") * _NC + lax.axis_index("c")
    base = wid * ROWS_PER_W

    pltpu.sync_copy(pos_hbm, pos_v)
    pltpu.sync_copy(idx_hbm.at[pl.ds(base, ROWS_PER_W)], idx_v)

    def chunk_body(g, carry):
        off = g * CHUNK
        pltpu.async_copy(
            table_hbm.at[idx_v.at[pl.ds(off, CHUNK)]], rows_v, sem
        ).wait()

        def s_body(s, c):
            p0 = pos_v[s, pl.ds(0, 16)]
            p1 = pos_v[s, pl.ds(16, 16)]
            for r in range(CHUNK_BATCH):
                q = r * SEQ + s
                rows_v[q, pl.ds(0, 16)] = rows_v[q, pl.ds(0, 16)] + p0
                rows_v[q, pl.ds(16, 16)] = rows_v[q, pl.ds(16, 16)] + p1
            return c

        lax.fori_loop(0, SEQ, s_body, 0)
        pltpu.sync_copy(
            rows_v, out_hbm.at[pl.ds(base + off, CHUNK), pl.ds(0, DIM)]
        )
        return carry

    lax.fori_loop(0, N_CHUNKS, chunk_body, 0)


_emb = functools.partial(
    pl.kernel,
    mesh=plsc.VectorSubcoreMesh(core_axis_name="c", subcore_axis_name="s"),
    out_type=jax.ShapeDtypeStruct((BATCH * SEQ, 128), jnp.float32),
    scratch_types=[
        pltpu.VMEM((ROWS_PER_W,), jnp.int32),
        pltpu.VMEM((CHUNK, DIM), jnp.float32),
        pltpu.VMEM((SEQ, DIM), jnp.float32),
        pltpu.SemaphoreType.DMA,
    ],
    compiler_params=pltpu.CompilerParams(use_tc_tiling_on_sc=False),
)(_emb_body)


def kernel(inputs, word_table, pos_table):
    flat_idx = inputs.reshape(-1).astype(jnp.int32) * 4
    table_padded = jnp.pad(word_table, ((0, 0), (0, 128 - DIM))).reshape(
        4 * VOCAB, DIM
    )
    out = _emb(flat_idx, table_padded, pos_table)
    return out[:, :DIM].reshape(inputs.shape[0], inputs.shape[1], DIM)
